# baseline (speedup 1.0000x reference)
import functools

import jax
import jax.numpy as jnp
from jax import lax
from jax.experimental import pallas as pl
from jax.experimental.pallas import tpu as pltpu
from jax.experimental.pallas import tpu_sc as plsc

_F = 16
_L = 16


def _make_sc_kernel(b, h, w, f):
    ng = w // _L
    mesh = plsc.VectorSubcoreMesh(core_axis_name="c", subcore_axis_name="s")

    @functools.partial(
        pl.kernel,
        out_type=jax.ShapeDtypeStruct((b, 2 * f, h, w), jnp.float32),
        mesh=mesh,
        scratch_types=[
            pltpu.VMEM((w,), jnp.float32),
            pltpu.VMEM((h, w), jnp.float32),
            pltpu.SemaphoreType.DMA,
        ],
    )
    def sc_kernel(col_t_hbm, row_t_hbm, out_hbm, vals_v, plane_v, sem):
        wid = lax.axis_index("s") * 2 + lax.axis_index("c")
        is_col = wid < f
        ch = wid % f

        @pl.when(is_col)
        def _():
            pltpu.sync_copy(col_t_hbm.at[ch], vals_v)
            for g in range(ng):
                v = vals_v[pl.ds(_L * g, _L)]
                for r in range(_L):
                    plane_v[r, pl.ds(_L * g, _L)] = v
            strip = plane_v.at[pl.ds(0, _L)]
            cps = [
                pltpu.async_copy(
                    strip, out_hbm.at[bi, wid, pl.ds(_L * gh, _L)], sem
                )
                for bi in range(b)
                for gh in range(h // _L)
            ]
            for cp in cps:
                cp.wait()

        @pl.when(jnp.logical_not(is_col))
        def _():
            pltpu.sync_copy(row_t_hbm.at[ch], vals_v)
            for gr in range(h // _L):
                vg = vals_v[pl.ds(_L * gr, _L)]
                for j in range(_L):
                    r = _L * gr + j
                    v = jnp.full((_L,), vg[j], jnp.float32)
                    for g in range(ng):
                        plane_v[r, pl.ds(_L * g, _L)] = v
            cps = [
                pltpu.async_copy(plane_v, out_hbm.at[bi, wid], sem)
                for bi in range(b)
            ]
            for cp in cps:
                cp.wait()

    return sc_kernel


def kernel(input, col_w, row_w):
    b = input.shape[0]
    h, w = input.shape[-2], input.shape[-1]
    f = col_w.shape[-1]
    col_t = col_w.T
    row_t = row_w.T
    return _make_sc_kernel(b, h, w, f)(col_t, row_t)


# speedup vs baseline: 1.1710x; 1.1710x over previous
import functools

import jax
import jax.numpy as jnp
from jax import lax
from jax.experimental import pallas as pl
from jax.experimental.pallas import tpu as pltpu
from jax.experimental.pallas import tpu_sc as plsc

_F = 16
_L = 16


def _make_sc_kernel(b, h, w, f):
    ng = w // _L
    mesh = plsc.VectorSubcoreMesh(core_axis_name="c", subcore_axis_name="s")

    @functools.partial(
        pl.kernel,
        out_type=jax.ShapeDtypeStruct((b, 2 * f, h, w), jnp.float32),
        mesh=mesh,
        scratch_types=[
            pltpu.VMEM((w,), jnp.float32),
            pltpu.VMEM((h, w), jnp.float32),
            pltpu.SemaphoreType.DMA,
        ],
    )
    def sc_kernel(col_t_hbm, row_t_hbm, out_hbm, vals_v, plane_v, sem):
        wid = lax.axis_index("s") * 2 + lax.axis_index("c")
        is_col = wid < f
        ch = wid % f

        @pl.when(is_col)
        def _():
            pltpu.sync_copy(col_t_hbm.at[ch], vals_v)
            for g in range(ng):
                v = vals_v[pl.ds(_L * g, _L)]
                for r in range(_L):
                    plane_v[r, pl.ds(_L * g, _L)] = v
            strip = plane_v.at[pl.ds(0, _L)]
            cps = [
                pltpu.async_copy(
                    strip, out_hbm.at[bi, wid, pl.ds(_L * gh, _L)], sem
                )
                for bi in range(b)
                for gh in range(h // _L)
            ]
            for cp in cps:
                cp.wait()

        @pl.when(jnp.logical_not(is_col))
        def _():
            pltpu.sync_copy(row_t_hbm.at[ch], vals_v)

            def _row_group(gr, carry):
                vg = vals_v[pl.ds(_L * gr, _L)]
                for j in range(_L):
                    r = _L * gr + j
                    v = jnp.full((_L,), vg[j], jnp.float32)
                    for g in range(ng):
                        plane_v[r, pl.ds(_L * g, _L)] = v
                return carry

            lax.fori_loop(0, h // _L, _row_group, 0)
            cps = [
                pltpu.async_copy(plane_v, out_hbm.at[bi, wid], sem)
                for bi in range(b)
            ]
            for cp in cps:
                cp.wait()

    return sc_kernel


def kernel(input, col_w, row_w):
    b = input.shape[0]
    h, w = input.shape[-2], input.shape[-1]
    f = col_w.shape[-1]
    col_t = col_w.T
    row_t = row_w.T
    return _make_sc_kernel(b, h, w, f)(col_t, row_t)


# speedup vs baseline: 1.2467x; 1.0647x over previous
import functools

import jax
import jax.numpy as jnp
from jax import lax
from jax.experimental import pallas as pl
from jax.experimental.pallas import tpu as pltpu
from jax.experimental.pallas import tpu_sc as plsc

_F = 16
_L = 16


def _make_sc_kernel(b, h, w, f):
    ng = w // _L
    mesh = plsc.VectorSubcoreMesh(core_axis_name="c", subcore_axis_name="s")

    @functools.partial(
        pl.kernel,
        out_type=jax.ShapeDtypeStruct((b, 2 * f, h, w), jnp.float32),
        mesh=mesh,
        scratch_types=[
            pltpu.VMEM((w,), jnp.float32),
            pltpu.VMEM((h, w), jnp.float32),
            pltpu.SemaphoreType.DMA,
        ],
    )
    def sc_kernel(col_t_hbm, row_t_hbm, out_hbm, vals_v, plane_v, sem):
        wid = lax.axis_index("s") * 2 + lax.axis_index("c")
        is_col = wid < f
        ch = wid % f

        @pl.when(is_col)
        def _():
            pltpu.sync_copy(col_t_hbm.at[ch], vals_v)

            def _strip_row(r, carry):
                for g in range(ng):
                    plane_v[r, pl.ds(_L * g, _L)] = vals_v[pl.ds(_L * g, _L)]
                return carry

            lax.fori_loop(0, _L, _strip_row, 0)
            strip = plane_v.at[pl.ds(0, _L)]
            cps = [
                pltpu.async_copy(
                    strip, out_hbm.at[bi, wid, pl.ds(_L * gh, _L)], sem
                )
                for bi in range(b)
                for gh in range(h // _L)
            ]
            for cp in cps:
                cp.wait()

        @pl.when(jnp.logical_not(is_col))
        def _():
            pltpu.sync_copy(row_t_hbm.at[ch], vals_v)

            def _row_group(gr, carry):
                vg = vals_v[pl.ds(_L * gr, _L)]
                vs = [jnp.full((_L,), vg[j], jnp.float32) for j in range(_L)]

                def _col_group(g, c2):
                    for j in range(_L):
                        plane_v[_L * gr + j, pl.ds(_L * g, _L)] = vs[j]
                    return c2

                lax.fori_loop(0, ng, _col_group, 0)
                return carry

            lax.fori_loop(0, h // _L, _row_group, 0)
            cps = [
                pltpu.async_copy(plane_v, out_hbm.at[bi, wid], sem)
                for bi in range(b)
            ]
            for cp in cps:
                cp.wait()

    return sc_kernel


def kernel(input, col_w, row_w):
    b = input.shape[0]
    h, w = input.shape[-2], input.shape[-1]
    f = col_w.shape[-1]
    col_t = col_w.T
    row_t = row_w.T
    return _make_sc_kernel(b, h, w, f)(col_t, row_t)


# speedup vs baseline: 1.2827x; 1.0288x over previous
import functools

import jax
import jax.numpy as jnp
from jax import lax
from jax.experimental import pallas as pl
from jax.experimental.pallas import tpu as pltpu
from jax.experimental.pallas import tpu_sc as plsc

_F = 16
_L = 16


def _make_sc_kernel(b, h, w, f):
    ng = w // _L
    mesh = plsc.VectorSubcoreMesh(core_axis_name="c", subcore_axis_name="s")

    @functools.partial(
        pl.kernel,
        out_type=jax.ShapeDtypeStruct((b, 2 * f, h, w), jnp.float32),
        mesh=mesh,
        scratch_types=[
            pltpu.VMEM((w,), jnp.float32),
            pltpu.VMEM((h, w), jnp.float32),
            pltpu.SemaphoreType.DMA,
        ],
    )
    def sc_kernel(col_t_hbm, row_t_hbm, out_hbm, vals_v, plane_v, sem):
        wid = lax.axis_index("s") * 2 + lax.axis_index("c")
        is_col = wid < f
        ch = wid % f

        def _fire(src_gr, gh):
            for bi in range(b):
                pltpu.async_copy(
                    plane_v.at[pl.ds(_L * src_gr, _L)],
                    out_hbm.at[bi, wid, pl.ds(_L * gh, _L)],
                    sem,
                )

        def _wait_strips():
            for bi in range(b):
                pltpu.make_async_copy(
                    plane_v.at[pl.ds(0, _L)],
                    out_hbm.at[bi, wid, pl.ds(0, _L)],
                    sem,
                ).wait()

        @pl.when(is_col)
        def _():
            pltpu.sync_copy(col_t_hbm.at[ch], vals_v)

            def _strip_row(r, carry):
                for g in range(ng):
                    plane_v[r, pl.ds(_L * g, _L)] = vals_v[pl.ds(_L * g, _L)]
                return carry

            lax.fori_loop(0, _L, _strip_row, 0)
            _fire(0, 0)

            def _pump(gh, carry):
                _fire(0, gh)
                _wait_strips()
                return carry

            lax.fori_loop(1, h // _L, _pump, 0)
            _wait_strips()

        @pl.when(jnp.logical_not(is_col))
        def _():
            pltpu.sync_copy(row_t_hbm.at[ch], vals_v)

            def _build_strip(gr):
                vg = vals_v[pl.ds(_L * gr, _L)]
                vs = [jnp.full((_L,), vg[j], jnp.float32) for j in range(_L)]

                def _col_group(g, c2):
                    for j in range(_L):
                        plane_v[_L * gr + j, pl.ds(_L * g, _L)] = vs[j]
                    return c2

                lax.fori_loop(0, ng, _col_group, 0)

            _build_strip(0)
            _fire(0, 0)

            def _pump(gr, carry):
                _build_strip(gr)
                _fire(gr, gr)
                _wait_strips()
                return carry

            lax.fori_loop(1, h // _L, _pump, 0)
            _wait_strips()

    return sc_kernel


def kernel(input, col_w, row_w):
    b = input.shape[0]
    h, w = input.shape[-2], input.shape[-1]
    f = col_w.shape[-1]
    col_t = col_w.T
    row_t = row_w.T
    return _make_sc_kernel(b, h, w, f)(col_t, row_t)


# speedup vs baseline: 1.2890x; 1.0049x over previous
import functools

import jax
import jax.numpy as jnp
from jax import lax
from jax.experimental import pallas as pl
from jax.experimental.pallas import tpu as pltpu
from jax.experimental.pallas import tpu_sc as plsc

_L = 16


def _make_sc_kernel(b, h, w, f):
    ng = w // _L
    mesh = plsc.VectorSubcoreMesh(core_axis_name="c", subcore_axis_name="s")

    @functools.partial(
        pl.kernel,
        out_type=jax.ShapeDtypeStruct((b, 2 * f, h, w), jnp.float32),
        mesh=mesh,
        scratch_types=[
            pltpu.VMEM((w,), jnp.float32),
            pltpu.VMEM((h, w), jnp.float32),
            pltpu.SemaphoreType.DMA,
        ],
    )
    def sc_kernel(col_t_hbm, row_t_hbm, out_hbm, vals_v, plane_v, sem):
        wid = lax.axis_index("s") * 2 + lax.axis_index("c")
        is_col = wid < f
        ch = wid % f

        def _fire(src_gr, gh):
            for bi in range(b):
                pltpu.async_copy(
                    plane_v.at[pl.ds(_L * src_gr, _L)],
                    out_hbm.at[bi, wid, pl.ds(_L * gh, _L)],
                    sem,
                )

        def _wait_strips():
            for bi in range(b):
                pltpu.make_async_copy(
                    plane_v.at[pl.ds(0, _L)],
                    out_hbm.at[bi, wid, pl.ds(0, _L)],
                    sem,
                ).wait()

        @pl.when(is_col)
        def _():
            pltpu.sync_copy(col_t_hbm.at[ch], vals_v)

            def _strip_row(r, carry):
                for g in range(ng):
                    plane_v[r, pl.ds(_L * g, _L)] = vals_v[pl.ds(_L * g, _L)]
                return carry

            lax.fori_loop(0, _L, _strip_row, 0)
            _fire(0, 0)

            def _pump(gh, carry):
                _fire(0, gh)
                _wait_strips()
                return carry

            lax.fori_loop(1, h // _L, _pump, 0)
            _wait_strips()

        @pl.when(jnp.logical_not(is_col))
        def _():
            pltpu.sync_copy(row_t_hbm.at[ch], vals_v)

            def _build_strip(gr):
                vg = vals_v[pl.ds(_L * gr, _L)]
                vs = [jnp.full((_L,), vg[j], jnp.float32) for j in range(_L)]

                def _col_group(g, c2):
                    for j in range(_L):
                        plane_v[_L * gr + j, pl.ds(_L * g, _L)] = vs[j]
                    return c2

                lax.fori_loop(0, ng, _col_group, 0)

            _build_strip(0)
            _fire(0, 0)

            def _pump(gr, carry):
                _build_strip(gr)
                _fire(gr, gr)
                _wait_strips()
                return carry

            lax.fori_loop(1, h // _L, _pump, 0)
            _wait_strips()

    return sc_kernel


def kernel(input, col_w, row_w):
    b = input.shape[0]
    h, w = input.shape[-2], input.shape[-1]
    f = col_w.shape[-1]
    col_t = col_w.T
    row_t = row_w.T
    return _make_sc_kernel(b, h, w, f)(col_t, row_t)
